# baseline (speedup 1.0000x reference)
import jax
import jax.numpy as jnp
from jax import lax
from jax.experimental import pallas as pl
from jax.experimental.pallas import tpu as pltpu
from jax.experimental.pallas import tpu_sc as plsc

D = 128
B = 16384
NC = 2
NS = 16
L = 16
NW = NC * NS
RPW = B // NW
C = 64
NCHUNK = RPW // C
NBUF = 3


def _rsqrt_newton(x):
    bits = plsc.bitcast(x, jnp.int32)
    y = plsc.bitcast(jnp.int32(0x5F3759DF) - (bits >> 1), jnp.float32)
    for _ in range(3):
        y = y * (1.5 - 0.5 * x * y * y)
    return y


def _body(head_hbm, rel_hbm, tail_hbm, ent_hbm, relemb_hbm, out_hbm,
          idx_h, idx_r, idx_t, outv, *scratch):
    bufs = tuple((scratch[3 * i], scratch[3 * i + 1], scratch[3 * i + 2],
                  scratch[3 * NBUF + 1 + i]) for i in range(NBUF))
    isem = scratch[3 * NBUF]

    wid = lax.axis_index("s") * NC + lax.axis_index("c")
    base = wid * RPW

    for dsc in (pltpu.async_copy(head_hbm.at[pl.ds(base, RPW)], idx_h, isem),
                pltpu.async_copy(rel_hbm.at[pl.ds(base, RPW)], idx_r, isem),
                pltpu.async_copy(tail_hbm.at[pl.ds(base, RPW)], idx_t, isem)):
        dsc.wait()

    def fire(c):
        hb, rb, tb, sem = bufs[c % NBUF]
        sl = pl.ds(c * C, C)
        return (
            pltpu.async_copy(ent_hbm.at[idx_h.at[sl]], hb, sem),
            pltpu.async_copy(relemb_hbm.at[idx_r.at[sl]], rb, sem),
            pltpu.async_copy(ent_hbm.at[idx_t.at[sl]], tb, sem),
        )

    descs = [fire(c) for c in range(min(NBUF, NCHUNK))]
    lane = lax.broadcasted_iota(jnp.int32, (L,), 0)
    for c in range(NCHUNK):
        for dsc in descs[c % NBUF]:
            dsc.wait()
        hb, rb, tb, _ = bufs[c % NBUF]
        for g in range(C // L):
            row = lane + g * L

            def dim_step(carry):
                acc, offs = carry
                vh = plsc.load_gather(hb, [row, offs])
                vr = plsc.load_gather(rb, [row, offs])
                vt = plsc.load_gather(tb, [row, offs])
                dif = (vh + vr) - vt
                return acc + dif * dif, (offs + 1) & (D - 1)

            def dim_step8(_, carry):
                for _u in range(8):
                    carry = dim_step(carry)
                return carry

            acc, _ = lax.fori_loop(0, D // 8, dim_step8,
                                   (jnp.zeros((L,), jnp.float32), lane))
            acc_s = jnp.maximum(acc, jnp.float32(1e-12))
            outv[pl.ds(c * C + g * L, L)] = acc * _rsqrt_newton(acc_s)
        if c + NBUF < NCHUNK:
            descs[c % NBUF] = fire(c + NBUF)

    pltpu.sync_copy(outv, out_hbm.at[pl.ds(base, RPW)])


@jax.jit
def _transe_sc(head, relation, tail, entity_embeddings, relation_embeddings):
    mesh = plsc.VectorSubcoreMesh(core_axis_name="c", subcore_axis_name="s",
                                  num_cores=NC, num_subcores=NS)
    scratch = (
        [pltpu.VMEM((RPW,), jnp.int32)] * 3
        + [pltpu.VMEM((RPW,), jnp.float32)]
        + [pltpu.VMEM((C, D), jnp.float32)] * (3 * NBUF)
        + [pltpu.SemaphoreType.DMA] * (1 + NBUF)
    )
    return pl.kernel(
        _body,
        out_type=jax.ShapeDtypeStruct((B,), jnp.float32),
        mesh=mesh,
        compiler_params=pltpu.CompilerParams(needs_layout_passes=False),
        scratch_types=scratch,
    )(head, relation, tail, entity_embeddings, relation_embeddings)


def kernel(head, relation, tail, entity_embeddings, relation_embeddings):
    return _transe_sc(head, relation, tail, entity_embeddings,
                      relation_embeddings)


# speedup vs baseline: 1.0678x; 1.0678x over previous
import jax
import jax.numpy as jnp
from jax import lax
from jax.experimental import pallas as pl
from jax.experimental.pallas import tpu as pltpu
from jax.experimental.pallas import tpu_sc as plsc

D = 128
B = 16384
NC = 2
NS = 16
L = 16
NW = NC * NS
RPW = B // NW
C = 64
NCHUNK = RPW // C
NBUF = 3


def _rsqrt_newton(x):
    bits = plsc.bitcast(x, jnp.int32)
    y = plsc.bitcast(jnp.int32(0x5F3759DF) - (bits >> 1), jnp.float32)
    for _ in range(3):
        y = y * (1.5 - 0.5 * x * y * y)
    return y


def _body(head_hbm, rel_hbm, tail_hbm, ent_hbm, relemb_hbm, out_hbm,
          idx_h, idx_r, idx_t, outv, *scratch):
    bufs = tuple((scratch[3 * i], scratch[3 * i + 1], scratch[3 * i + 2],
                  scratch[3 * NBUF + 1 + i]) for i in range(NBUF))
    isem = scratch[3 * NBUF]

    wid = lax.axis_index("s") * NC + lax.axis_index("c")
    base = wid * RPW

    for dsc in (pltpu.async_copy(head_hbm.at[pl.ds(base, RPW)], idx_h, isem),
                pltpu.async_copy(rel_hbm.at[pl.ds(base, RPW)], idx_r, isem),
                pltpu.async_copy(tail_hbm.at[pl.ds(base, RPW)], idx_t, isem)):
        dsc.wait()

    def fire(c):
        hb, rb, tb, sem = bufs[c % NBUF]
        sl = pl.ds(c * C, C)
        return (
            pltpu.async_copy(ent_hbm.at[idx_h.at[sl]], hb, sem),
            pltpu.async_copy(relemb_hbm.at[idx_r.at[sl]], rb, sem),
            pltpu.async_copy(ent_hbm.at[idx_t.at[sl]], tb, sem),
        )

    descs = [fire(c) for c in range(min(NBUF, NCHUNK))]
    lane = lax.broadcasted_iota(jnp.int32, (L,), 0)
    for c in range(NCHUNK):
        for dsc in descs[c % NBUF]:
            dsc.wait()
        hb, rb, tb, _ = bufs[c % NBUF]
        for g in range(C // L):
            row = lane + g * L

            def dim_step(carry):
                acc, offs = carry
                vh = plsc.load_gather(hb, [row, offs])
                vr = plsc.load_gather(rb, [row, offs])
                vt = plsc.load_gather(tb, [row, offs])
                dif = (vh + vr) - vt
                return acc + dif * dif, (offs + 1) & (D - 1)

            def dim_step2(_, carry):
                for _u in range(2):
                    carry = dim_step(carry)
                return carry

            acc, _ = lax.fori_loop(0, D // 2, dim_step2,
                                   (jnp.zeros((L,), jnp.float32), lane))
            acc_s = jnp.maximum(acc, jnp.float32(1e-12))
            outv[pl.ds(c * C + g * L, L)] = acc * _rsqrt_newton(acc_s)
        if c + NBUF < NCHUNK:
            descs[c % NBUF] = fire(c + NBUF)

    pltpu.sync_copy(outv, out_hbm.at[pl.ds(base, RPW)])


@jax.jit
def _transe_sc(head, relation, tail, entity_embeddings, relation_embeddings):
    mesh = plsc.VectorSubcoreMesh(core_axis_name="c", subcore_axis_name="s",
                                  num_cores=NC, num_subcores=NS)
    scratch = (
        [pltpu.VMEM((RPW,), jnp.int32)] * 3
        + [pltpu.VMEM((RPW,), jnp.float32)]
        + [pltpu.VMEM((C, D), jnp.float32)] * (3 * NBUF)
        + [pltpu.SemaphoreType.DMA] * (1 + NBUF)
    )
    return pl.kernel(
        _body,
        out_type=jax.ShapeDtypeStruct((B,), jnp.float32),
        mesh=mesh,
        compiler_params=pltpu.CompilerParams(needs_layout_passes=False),
        scratch_types=scratch,
    )(head, relation, tail, entity_embeddings, relation_embeddings)


def kernel(head, relation, tail, entity_embeddings, relation_embeddings):
    return _transe_sc(head, relation, tail, entity_embeddings,
                      relation_embeddings)


# speedup vs baseline: 1.0946x; 1.0250x over previous
import jax
import jax.numpy as jnp
from jax import lax
from jax.experimental import pallas as pl
from jax.experimental.pallas import tpu as pltpu
from jax.experimental.pallas import tpu_sc as plsc

D = 128
B = 16384
NC = 2
NS = 16
L = 16
NW = NC * NS
RPW = B // NW
C = 64
NCHUNK = RPW // C
NBUF = 3


def _rsqrt_newton(x):
    bits = plsc.bitcast(x, jnp.int32)
    y = plsc.bitcast(jnp.int32(0x5F3759DF) - (bits >> 1), jnp.float32)
    for _ in range(3):
        y = y * (1.5 - 0.5 * x * y * y)
    return y


def _body(head_hbm, rel_hbm, tail_hbm, ent_hbm, relemb_hbm, out_hbm,
          idx_h, idx_r, idx_t, outv, *scratch):
    bufs = tuple((scratch[3 * i], scratch[3 * i + 1], scratch[3 * i + 2],
                  scratch[3 * NBUF + 1 + i]) for i in range(NBUF))
    isem = scratch[3 * NBUF]

    wid = lax.axis_index("s") * NC + lax.axis_index("c")
    base = wid * RPW

    d_a = (pltpu.async_copy(head_hbm.at[pl.ds(base, C)], idx_h.at[pl.ds(0, C)], isem),
           pltpu.async_copy(rel_hbm.at[pl.ds(base, C)], idx_r.at[pl.ds(0, C)], isem),
           pltpu.async_copy(tail_hbm.at[pl.ds(base, C)], idx_t.at[pl.ds(0, C)], isem))
    d_b = (pltpu.async_copy(head_hbm.at[pl.ds(base + C, RPW - C)], idx_h.at[pl.ds(C, RPW - C)], isem),
           pltpu.async_copy(rel_hbm.at[pl.ds(base + C, RPW - C)], idx_r.at[pl.ds(C, RPW - C)], isem),
           pltpu.async_copy(tail_hbm.at[pl.ds(base + C, RPW - C)], idx_t.at[pl.ds(C, RPW - C)], isem))

    def fire(c):
        hb, rb, tb, sem = bufs[c % NBUF]
        sl = pl.ds(c * C, C)
        return (
            pltpu.async_copy(ent_hbm.at[idx_h.at[sl]], hb, sem),
            pltpu.async_copy(relemb_hbm.at[idx_r.at[sl]], rb, sem),
            pltpu.async_copy(ent_hbm.at[idx_t.at[sl]], tb, sem),
        )

    for dsc in d_a:
        dsc.wait()
    descs = [fire(0)]
    for dsc in d_b:
        dsc.wait()
    descs += [fire(c) for c in range(1, min(NBUF, NCHUNK))]
    lane = lax.broadcasted_iota(jnp.int32, (L,), 0)
    for c in range(NCHUNK):
        for dsc in descs[c % NBUF]:
            dsc.wait()
        hb, rb, tb, _ = bufs[c % NBUF]
        for g in range(C // L):
            row = lane + g * L

            def dim_step(carry):
                acc, offs = carry
                vh = plsc.load_gather(hb, [row, offs])
                vr = plsc.load_gather(rb, [row, offs])
                vt = plsc.load_gather(tb, [row, offs])
                dif = (vh + vr) - vt
                return acc + dif * dif, (offs + 1) & (D - 1)

            def dim_step4(_, carry):
                for _u in range(4):
                    carry = dim_step(carry)
                return carry

            acc, _ = lax.fori_loop(0, D // 4, dim_step4,
                                   (jnp.zeros((L,), jnp.float32), lane))
            acc_s = jnp.maximum(acc, jnp.float32(1e-12))
            outv[pl.ds(c * C + g * L, L)] = acc * _rsqrt_newton(acc_s)
        if c + NBUF < NCHUNK:
            descs[c % NBUF] = fire(c + NBUF)

    pltpu.sync_copy(outv, out_hbm.at[pl.ds(base, RPW)])


@jax.jit
def _transe_sc(head, relation, tail, entity_embeddings, relation_embeddings):
    mesh = plsc.VectorSubcoreMesh(core_axis_name="c", subcore_axis_name="s",
                                  num_cores=NC, num_subcores=NS)
    scratch = (
        [pltpu.VMEM((RPW,), jnp.int32)] * 3
        + [pltpu.VMEM((RPW,), jnp.float32)]
        + [pltpu.VMEM((C, D), jnp.float32)] * (3 * NBUF)
        + [pltpu.SemaphoreType.DMA] * (1 + NBUF)
    )
    return pl.kernel(
        _body,
        out_type=jax.ShapeDtypeStruct((B,), jnp.float32),
        mesh=mesh,
        compiler_params=pltpu.CompilerParams(needs_layout_passes=False),
        scratch_types=scratch,
    )(head, relation, tail, entity_embeddings, relation_embeddings)


def kernel(head, relation, tail, entity_embeddings, relation_embeddings):
    return _transe_sc(head, relation, tail, entity_embeddings,
                      relation_embeddings)
